# baseline (speedup 1.0000x reference)
import jax
import jax.numpy as jnp
import numpy as np
from jax.experimental import pallas as pl
from jax.experimental.pallas import tpu as pltpu

_B, _C, _H, _W = 16, 1024, 32, 32
_HW = _H * _W
_HH = 16
_T = _HH * _W
_RC = 16

_KS0 = np.uint32(0)
_KS1 = np.uint32(42)
_KS2 = np.uint32(_KS0 ^ _KS1 ^ np.uint32(0x1BD11BDA))
_ROT = ((13, 15, 26, 6), (17, 29, 16, 24))


def _rounds(x0, x1, rs):
    for r in rs:
        x0 = x0 + x1
        x1 = (x1 << jnp.uint32(r)) | (x1 >> jnp.uint32(32 - r))
        x1 = x0 ^ x1
    return x0, x1


def _gumbel_chunk(cnt):
    x0 = jnp.zeros(cnt.shape, jnp.uint32) + jnp.uint32(_KS0)
    x1 = cnt + jnp.uint32(_KS1)
    x0, x1 = _rounds(x0, x1, _ROT[0])
    x0 = x0 + jnp.uint32(_KS1)
    x1 = x1 + jnp.uint32(_KS2 + np.uint32(1))
    x0, x1 = _rounds(x0, x1, _ROT[1])
    x0 = x0 + jnp.uint32(_KS2)
    x1 = x1 + jnp.uint32(_KS0 + np.uint32(2))
    x0, x1 = _rounds(x0, x1, _ROT[0])
    x0 = x0 + jnp.uint32(_KS0)
    x1 = x1 + jnp.uint32(_KS1 + np.uint32(3))
    x0, x1 = _rounds(x0, x1, _ROT[1])
    x0 = x0 + jnp.uint32(_KS1)
    x1 = x1 + jnp.uint32(_KS2 + np.uint32(4))
    x0, x1 = _rounds(x0, x1, _ROT[0])
    x0 = x0 + jnp.uint32(_KS2)
    x1 = x1 + jnp.uint32(_KS0 + np.uint32(5))

    bits = x0 ^ x1
    fb = (bits >> jnp.uint32(9)) | jnp.uint32(0x3F800000)
    f = jax.lax.bitcast_convert_type(fb, jnp.float32) - jnp.float32(1.0)
    tiny = jnp.float32(np.finfo(np.float32).tiny)
    span = jnp.float32(np.float32(1.0) - np.finfo(np.float32).tiny)
    u = jnp.maximum(tiny, f * span + tiny)
    return -jnp.log(-jnp.log(u))


def _onehot_argmax_kernel(x_ref, o_ref, xc_ref):
    b = pl.program_id(0).astype(jnp.uint32)
    j = pl.program_id(1).astype(jnp.uint32)
    base = b * jnp.uint32(_C * _HW) + j * jnp.uint32(_T)

    xc_ref[...] = x_ref[0].reshape(_C, _T)

    k = jax.lax.broadcasted_iota(jnp.uint32, (_RC, _T), 0)
    t = jax.lax.broadcasted_iota(jnp.uint32, (_RC, _T), 1)
    cnt0 = base + k * jnp.uint32(_HW) + t
    krow = k.astype(jnp.int32)

    def body(i, carry):
        acc_val, acc_row = carry
        for half in range(2):
            c0 = i * 2 + half
            cnt = cnt0 + (c0 * (_RC * _HW)).astype(jnp.uint32)
            g = _gumbel_chunk(cnt)
            s = xc_ref[pl.ds(c0 * _RC, _RC), :] + g
            pred = s > acc_val
            rows = krow + c0 * _RC
            acc_val = jnp.where(pred, s, acc_val)
            acc_row = jnp.where(pred, rows, acc_row)
        return acc_val, acc_row

    init = (jnp.full((_RC, _T), -jnp.inf, jnp.float32),
            jnp.zeros((_RC, _T), jnp.int32))
    acc_val, acc_row = jax.lax.fori_loop(0, _C // (2 * _RC), body, init)

    maxv = jnp.max(acc_val, axis=0, keepdims=True)
    cand = jnp.where(acc_val == maxv, acc_row, jnp.int32(2**31 - 1))
    idx = jnp.min(cand, axis=0, keepdims=True)

    iota = jax.lax.broadcasted_iota(jnp.int32, (_C, _T), 0)
    onehot = (iota == idx).astype(jnp.float32)
    o_ref[0] = onehot.reshape(_C, _HH, _W)


def kernel(x):
    out = pl.pallas_call(
        _onehot_argmax_kernel,
        grid=(_B, _H // _HH),
        in_specs=[
            pl.BlockSpec((1, _C, _HH, _W), lambda b, j: (b, 0, j, 0)),
        ],
        out_specs=pl.BlockSpec((1, _C, _HH, _W), lambda b, j: (b, 0, j, 0)),
        out_shape=jax.ShapeDtypeStruct((_B, _C, _H, _W), jnp.float32),
        scratch_shapes=[pltpu.VMEM((_C, _T), jnp.float32)],
    )(x)
    return out


# speedup vs baseline: 1.3513x; 1.3513x over previous
import jax
import jax.numpy as jnp
import numpy as np
from jax.experimental import pallas as pl
from jax.experimental.pallas import tpu as pltpu

_B, _C, _H, _W = 16, 1024, 32, 32
_HW = _H * _W
_T = _HW
_RC = 8

_KS0 = np.uint32(0)
_KS1 = np.uint32(42)
_KS2 = np.uint32(_KS0 ^ _KS1 ^ np.uint32(0x1BD11BDA))
_ROT = ((13, 15, 26, 6), (17, 29, 16, 24))


def _rounds(x0, x1, rs):
    for r in rs:
        x0 = x0 + x1
        x1 = (x1 << jnp.uint32(r)) | (x1 >> jnp.uint32(32 - r))
        x1 = x0 ^ x1
    return x0, x1


def _gumbel_chunk(cnt):
    x0 = jnp.zeros(cnt.shape, jnp.uint32) + jnp.uint32(_KS0)
    x1 = cnt + jnp.uint32(_KS1)
    x0, x1 = _rounds(x0, x1, _ROT[0])
    x0 = x0 + jnp.uint32(_KS1)
    x1 = x1 + jnp.uint32(_KS2 + np.uint32(1))
    x0, x1 = _rounds(x0, x1, _ROT[1])
    x0 = x0 + jnp.uint32(_KS2)
    x1 = x1 + jnp.uint32(_KS0 + np.uint32(2))
    x0, x1 = _rounds(x0, x1, _ROT[0])
    x0 = x0 + jnp.uint32(_KS0)
    x1 = x1 + jnp.uint32(_KS1 + np.uint32(3))
    x0, x1 = _rounds(x0, x1, _ROT[1])
    x0 = x0 + jnp.uint32(_KS1)
    x1 = x1 + jnp.uint32(_KS2 + np.uint32(4))
    x0, x1 = _rounds(x0, x1, _ROT[0])
    x0 = x0 + jnp.uint32(_KS2)
    x1 = x1 + jnp.uint32(_KS0 + np.uint32(5))

    bits = x0 ^ x1
    fb = (bits >> jnp.uint32(9)) | jnp.uint32(0x3F800000)
    f = jax.lax.bitcast_convert_type(fb, jnp.float32) - jnp.float32(1.0)
    tiny = jnp.float32(np.finfo(np.float32).tiny)
    span = jnp.float32(np.float32(1.0) - np.finfo(np.float32).tiny)
    u = jnp.maximum(tiny, f * span + tiny)
    return -jnp.log(-jnp.log(u))


def _onehot_argmax_kernel(x_ref, o_ref):
    b = pl.program_id(0).astype(jnp.uint32)
    base = b * jnp.uint32(_C * _HW)

    k = jax.lax.broadcasted_iota(jnp.uint32, (_RC, _T), 0)
    t = jax.lax.broadcasted_iota(jnp.uint32, (_RC, _T), 1)
    cnt0 = base + k * jnp.uint32(_HW) + t
    krow = k.astype(jnp.int32)

    def body(i, carry):
        acc_val, acc_row = carry
        for half in range(2):
            c0 = i * 2 + half
            cnt = cnt0 + (c0 * (_RC * _HW)).astype(jnp.uint32)
            g = _gumbel_chunk(cnt)
            s = x_ref[0, pl.ds(c0 * _RC, _RC), :] + g
            pred = s > acc_val
            rows = krow + c0 * _RC
            acc_val = jnp.where(pred, s, acc_val)
            acc_row = jnp.where(pred, rows, acc_row)
        return acc_val, acc_row

    init = (jnp.full((_RC, _T), -jnp.inf, jnp.float32),
            jnp.zeros((_RC, _T), jnp.int32))
    acc_val, acc_row = jax.lax.fori_loop(0, _C // (2 * _RC), body, init)

    maxv = jnp.max(acc_val, axis=0, keepdims=True)
    cand = jnp.where(acc_val == maxv, acc_row, jnp.int32(2**31 - 1))
    idx = jnp.min(cand, axis=0, keepdims=True)

    iota = jax.lax.broadcasted_iota(jnp.int32, (_C, _T), 0)
    onehot = (iota == idx).astype(jnp.float32)
    o_ref[0] = onehot.reshape(_C, _H, _W)


def kernel(x):
    xr = x.reshape(_B, _C, _HW)
    out = pl.pallas_call(
        _onehot_argmax_kernel,
        grid=(_B,),
        in_specs=[
            pl.BlockSpec((1, _C, _T), lambda b: (b, 0, 0)),
        ],
        out_specs=pl.BlockSpec((1, _C, _H, _W), lambda b: (b, 0, 0, 0)),
        out_shape=jax.ShapeDtypeStruct((_B, _C, _H, _W), jnp.float32),
    )(xr)
    return out


# speedup vs baseline: 1.9452x; 1.4395x over previous
import jax
import jax.numpy as jnp
import numpy as np
from jax.experimental import pallas as pl
from jax.experimental.pallas import tpu as pltpu

_B, _C, _H, _W = 16, 1024, 32, 32
_HW = _H * _W
_T = _HW
_RC = 8

_KS0 = np.uint32(0)
_KS1 = np.uint32(42)
_KS2 = np.uint32(_KS0 ^ _KS1 ^ np.uint32(0x1BD11BDA))
_ROT = ((13, 15, 26, 6), (17, 29, 16, 24))


def _rounds(x0, x1, rs):
    for r in rs:
        x0 = x0 + x1
        x1 = (x1 << jnp.uint32(r)) | (x1 >> jnp.uint32(32 - r))
        x1 = x0 ^ x1
    return x0, x1


def _gumbel_chunk(cnt):
    x0 = jnp.zeros(cnt.shape, jnp.uint32) + jnp.uint32(_KS0)
    x1 = cnt + jnp.uint32(_KS1)
    x0, x1 = _rounds(x0, x1, _ROT[0])
    x0 = x0 + jnp.uint32(_KS1)
    x1 = x1 + jnp.uint32(_KS2 + np.uint32(1))
    x0, x1 = _rounds(x0, x1, _ROT[1])
    x0 = x0 + jnp.uint32(_KS2)
    x1 = x1 + jnp.uint32(_KS0 + np.uint32(2))
    x0, x1 = _rounds(x0, x1, _ROT[0])
    x0 = x0 + jnp.uint32(_KS0)
    x1 = x1 + jnp.uint32(_KS1 + np.uint32(3))
    x0, x1 = _rounds(x0, x1, _ROT[1])
    x0 = x0 + jnp.uint32(_KS1)
    x1 = x1 + jnp.uint32(_KS2 + np.uint32(4))
    x0, x1 = _rounds(x0, x1, _ROT[0])
    x0 = x0 + jnp.uint32(_KS2)
    x1 = x1 + jnp.uint32(_KS0 + np.uint32(5))

    bits = x0 ^ x1
    fb = (bits >> jnp.uint32(9)) | jnp.uint32(0x3F800000)
    f = jax.lax.bitcast_convert_type(fb, jnp.float32) - jnp.float32(1.0)
    tiny = jnp.float32(np.finfo(np.float32).tiny)
    span = jnp.float32(np.float32(1.0) - np.finfo(np.float32).tiny)
    u = jnp.maximum(tiny, f * span + tiny)
    return -jnp.log(-jnp.log(u))


def _onehot_argmax_kernel(x_ref, o_ref):
    b = pl.program_id(0).astype(jnp.uint32)
    base = b * jnp.uint32(_C * _HW)

    k = jax.lax.broadcasted_iota(jnp.uint32, (_RC, _T), 0)
    t = jax.lax.broadcasted_iota(jnp.uint32, (_RC, _T), 1)
    cnt0 = base + k * jnp.uint32(_HW) + t
    krow = k.astype(jnp.int32)

    def body(i, carry):
        acc_val, acc_row = carry
        for half in range(2):
            c0 = i * 2 + half
            cnt = cnt0 + (c0 * (_RC * _HW)).astype(jnp.uint32)
            g = _gumbel_chunk(cnt)
            s = x_ref[0, pl.ds(c0 * _RC, _RC), :] + g
            pred = s > acc_val
            rows = krow + c0 * _RC
            acc_val = jnp.where(pred, s, acc_val)
            acc_row = jnp.where(pred, rows, acc_row)
        return acc_val, acc_row

    init = (jnp.full((_RC, _T), -jnp.inf, jnp.float32),
            jnp.zeros((_RC, _T), jnp.int32))
    acc_val, acc_row = jax.lax.fori_loop(0, _C // (2 * _RC), body, init)

    maxv = jnp.max(acc_val, axis=0, keepdims=True)
    cand = jnp.where(acc_val == maxv, acc_row, jnp.int32(2**31 - 1))
    idx = jnp.min(cand, axis=0, keepdims=True)

    iota = jax.lax.broadcasted_iota(jnp.int32, (_C, _T), 0)
    o_ref[0] = (iota == idx).astype(jnp.float32)


def kernel(x):
    xr = x.reshape(_B, _C, _HW)
    out = pl.pallas_call(
        _onehot_argmax_kernel,
        grid=(_B,),
        in_specs=[
            pl.BlockSpec((1, _C, _T), lambda b: (b, 0, 0)),
        ],
        out_specs=pl.BlockSpec((1, _C, _T), lambda b: (b, 0, 0)),
        out_shape=jax.ShapeDtypeStruct((_B, _C, _HW), jnp.float32),
    )(xr)
    return out.reshape(_B, _C, _H, _W)


# speedup vs baseline: 1.9813x; 1.0186x over previous
import jax
import jax.numpy as jnp
import numpy as np
from jax.experimental import pallas as pl
from jax.experimental.pallas import tpu as pltpu

_B, _C, _H, _W = 16, 1024, 32, 32
_HW = _H * _W
_T = _HW
_RC = 8

_KS0 = np.uint32(0)
_KS1 = np.uint32(42)
_KS2 = np.uint32(_KS0 ^ _KS1 ^ np.uint32(0x1BD11BDA))
_ROT = ((13, 15, 26, 6), (17, 29, 16, 24))


def _rounds(x0, x1, rs):
    for r in rs:
        x0 = x0 + x1
        x1 = (x1 << jnp.uint32(r)) | (x1 >> jnp.uint32(32 - r))
        x1 = x0 ^ x1
    return x0, x1


def _gumbel_chunk(cnt):
    x0 = jnp.zeros(cnt.shape, jnp.uint32) + jnp.uint32(_KS0)
    x1 = cnt + jnp.uint32(_KS1)
    x0, x1 = _rounds(x0, x1, _ROT[0])
    x0 = x0 + jnp.uint32(_KS1)
    x1 = x1 + jnp.uint32(_KS2 + np.uint32(1))
    x0, x1 = _rounds(x0, x1, _ROT[1])
    x0 = x0 + jnp.uint32(_KS2)
    x1 = x1 + jnp.uint32(_KS0 + np.uint32(2))
    x0, x1 = _rounds(x0, x1, _ROT[0])
    x0 = x0 + jnp.uint32(_KS0)
    x1 = x1 + jnp.uint32(_KS1 + np.uint32(3))
    x0, x1 = _rounds(x0, x1, _ROT[1])
    x0 = x0 + jnp.uint32(_KS1)
    x1 = x1 + jnp.uint32(_KS2 + np.uint32(4))
    x0, x1 = _rounds(x0, x1, _ROT[0])
    x0 = x0 + jnp.uint32(_KS2)
    x1 = x1 + jnp.uint32(_KS0 + np.uint32(5))

    bits = x0 ^ x1
    fb = (bits >> jnp.uint32(9)) | jnp.uint32(0x3F800000)
    f = jax.lax.bitcast_convert_type(fb, jnp.float32) - jnp.float32(1.0)
    tiny = jnp.float32(np.finfo(np.float32).tiny)
    span = jnp.float32(np.float32(1.0) - np.finfo(np.float32).tiny)
    u = jnp.maximum(tiny, f * span + tiny)
    return -jnp.log(-jnp.log(u))


def _onehot_argmax_kernel(x_ref, o_ref):
    b = pl.program_id(0).astype(jnp.uint32)
    base = b * jnp.uint32(_C * _HW)

    k = jax.lax.broadcasted_iota(jnp.uint32, (_RC, _T), 0)
    t = jax.lax.broadcasted_iota(jnp.uint32, (_RC, _T), 1)
    cnt0 = base + k * jnp.uint32(_HW) + t
    krow = k.astype(jnp.int32)

    def body(i, carry):
        acc_val, acc_row = carry
        for half in range(4):
            c0 = i * 4 + half
            cnt = cnt0 + (c0 * (_RC * _HW)).astype(jnp.uint32)
            g = _gumbel_chunk(cnt)
            s = x_ref[0, pl.ds(c0 * _RC, _RC), :] + g
            pred = s > acc_val
            rows = krow + c0 * _RC
            acc_val = jnp.where(pred, s, acc_val)
            acc_row = jnp.where(pred, rows, acc_row)
        return acc_val, acc_row

    init = (jnp.full((_RC, _T), -jnp.inf, jnp.float32),
            jnp.zeros((_RC, _T), jnp.int32))
    acc_val, acc_row = jax.lax.fori_loop(0, _C // (4 * _RC), body, init)

    maxv = jnp.max(acc_val, axis=0, keepdims=True)
    cand = jnp.where(acc_val == maxv, acc_row, jnp.int32(2**31 - 1))
    idx = jnp.min(cand, axis=0, keepdims=True)

    iota = jax.lax.broadcasted_iota(jnp.int32, (_C, _T), 0)
    o_ref[0] = (iota == idx).astype(jnp.float32)


def kernel(x):
    xr = x.reshape(_B, _C, _HW)
    out = pl.pallas_call(
        _onehot_argmax_kernel,
        grid=(_B,),
        in_specs=[
            pl.BlockSpec((1, _C, _T), lambda b: (b, 0, 0)),
        ],
        out_specs=pl.BlockSpec((1, _C, _T), lambda b: (b, 0, 0)),
        out_shape=jax.ShapeDtypeStruct((_B, _C, _HW), jnp.float32),
    )(xr)
    return out.reshape(_B, _C, _H, _W)
